# baseline (speedup 1.0000x reference)
import jax
import jax.numpy as jnp
from jax.experimental import pallas as pl
from jax.experimental.pallas import tpu as pltpu

N3 = 300000
N = N3 // 3
D = 128
B = 4000
NZ = N // B
SC_ = 1000
SR_ = N // SC_


def _stream_body(z_ref, zp_ref, zn_ref, wz_ref, wp_ref, wn_ref,
                 dp_ref, dn_ref, s2z_ref, s2p_ref, s2n_ref,
                 s_ref, cnt_ref):
    g = pl.program_id(0)

    @pl.when(g == 0)
    def _init():
        s_ref[...] = jnp.zeros_like(s_ref)
        cnt_ref[...] = jnp.zeros_like(cnt_ref)

    z = z_ref[...]
    zp = zp_ref[...]
    zn = zn_ref[...]
    wz = wz_ref[0]
    wp = wp_ref[0]
    wn = wn_ref[0]

    zb = z.astype(jnp.bfloat16)
    zpb = zp.astype(jnp.bfloat16)
    znb = zn.astype(jnp.bfloat16)

    ones_col = jnp.ones((D, 1), jnp.bfloat16)

    def colsum(w, x):
        return jax.lax.dot_general(
            w, x, (((0,), (0,)), ((), ())),
            preferred_element_type=jnp.float32)

    def rowred(e, rhs):
        return jax.lax.dot_general(
            e, rhs, (((1,), (0,)), ((), ())),
            preferred_element_type=jnp.float32)

    @pl.when(g < NZ)
    def _phase_a():
        dp_ref[0] = rowred(zb * zpb, ones_col)
        dn_ref[0] = rowred(zb * znb, -ones_col)
        s_ref[...] += colsum(wz, z) + colsum(wp, zp) + colsum(wn, zn)
        cnt_ref[...] += colsum(wz, wz) + colsum(wp, wp) + colsum(wn, wn)

    @pl.when(g >= NZ)
    def _phase_b():
        denom = jnp.maximum(cnt_ref[...], 1.0)
        mu = s_ref[...] / denom
        m = jnp.sum(mu * mu)
        mu_col = mu.reshape(D, 1).astype(jnp.bfloat16) * jnp.bfloat16(-2.0)

        def stage(o_ref, xb, w):
            q = rowred(xb * xb, ones_col)
            r = rowred(xb, mu_col)
            o_ref[0] = w * (q + r + m)

        stage(s2z_ref, zb, wz)
        stage(s2p_ref, zpb, wp)
        stage(s2n_ref, znb, wn)


def _tail_body(dp_ref, dn_ref, s2z_ref, s2p_ref, s2n_ref, o_ref):
    def logsig_sum(x):
        return jnp.sum(jnp.minimum(x, 0.0) - jnp.log1p(jnp.exp(-jnp.abs(x))))

    def norm_sum(x):
        return jnp.sum(jnp.sqrt(jnp.maximum(x, 0.0)))

    posneg = logsig_sum(dp_ref[...]) + logsig_sum(dn_ref[...])
    coag = norm_sum(s2z_ref[...]) + norm_sum(s2p_ref[...]) + norm_sum(s2n_ref[...])
    sig = 1.0 / (1.0 + jnp.exp(-coag))
    total = -posneg / N + sig - 0.5
    o_ref[...] = jnp.full((1, 1), total, dtype=jnp.float32)


def kernel(out, mask):
    w = mask.astype(jnp.float32).reshape(3 * NZ, B, 1)

    def omap(t):
        return lambda g: (t * NZ + g % NZ, 0)

    def wmap(t):
        return lambda g: (t * NZ + g % NZ, 0, 0)

    def amap(g):
        return (jnp.minimum(g, NZ), 0, 0)

    def bmap(g):
        return (jnp.where(g < NZ, NZ, g - NZ), 0, 0)

    stg = jax.ShapeDtypeStruct((NZ + 1, B, 1), jnp.float32)
    stg_spec = pl.BlockSpec((1, B, 1), amap)
    stg_spec_b = pl.BlockSpec((1, B, 1), bmap)

    dp, dn, s2z, s2p, s2n = pl.pallas_call(
        _stream_body,
        grid=(2 * NZ,),
        in_specs=[
            pl.BlockSpec((B, D), omap(0)),
            pl.BlockSpec((B, D), omap(1)),
            pl.BlockSpec((B, D), omap(2)),
            pl.BlockSpec((1, B, 1), wmap(0)),
            pl.BlockSpec((1, B, 1), wmap(1)),
            pl.BlockSpec((1, B, 1), wmap(2)),
        ],
        out_specs=[stg_spec, stg_spec, stg_spec_b, stg_spec_b, stg_spec_b],
        out_shape=[stg, stg, stg, stg, stg],
        scratch_shapes=[
            pltpu.VMEM((1, D), jnp.float32),
            pltpu.VMEM((1, 1), jnp.float32),
        ],
        compiler_params=pltpu.CompilerParams(
            dimension_semantics=("arbitrary",),
        ),
    )(out, out, out, w, w, w)

    def flat(x):
        return x[:NZ].reshape(SR_, SC_)

    res = pl.pallas_call(
        _tail_body,
        out_specs=pl.BlockSpec((1, 1), lambda: (0, 0)),
        out_shape=jax.ShapeDtypeStruct((1, 1), jnp.float32),
    )(flat(dp), flat(dn), flat(s2z), flat(s2p), flat(s2n))
    return res[0, 0]


# speedup vs baseline: 4.7277x; 4.7277x over previous
import jax
import jax.numpy as jnp
from jax.experimental import pallas as pl
from jax.experimental.pallas import tpu as pltpu

N3 = 300000
N = N3 // 3
D = 128
B = 4000
NZ = N // B


def _body(z_ref, zp_ref, zn_ref, wz_ref, wp_ref, wn_ref, o_ref,
          s_ref, sc_ref):
    g = pl.program_id(0)

    @pl.when(g == 0)
    def _init():
        s_ref[...] = jnp.zeros_like(s_ref)
        sc_ref[0] = 0.0
        sc_ref[1] = 0.0
        sc_ref[2] = 0.0
        sc_ref[3] = 0.0

    zb = z_ref[...].astype(jnp.bfloat16)
    zpb = zp_ref[...].astype(jnp.bfloat16)
    znb = zn_ref[...].astype(jnp.bfloat16)
    wz = wz_ref[0]
    wp = wp_ref[0]
    wn = wn_ref[0]

    ones_row = jnp.ones((1, D), jnp.bfloat16)

    def lanered(v, e):
        return jax.lax.dot_general(
            v, e, (((1,), (1,)), ((), ())),
            preferred_element_type=jnp.float32)

    def colsum(w, x):
        return jax.lax.dot_general(
            w, x, (((1,), (0,)), ((), ())),
            preferred_element_type=jnp.float32)

    def logsig_sum(x):
        return jnp.sum(jnp.minimum(x, 0.0) - jnp.log1p(jnp.exp(-jnp.abs(x))))

    @pl.when(g < NZ)
    def _phase_a():
        dp = lanered(ones_row, zb * zpb)
        dn = lanered(ones_row, zb * znb)
        sc_ref[0] += logsig_sum(dp)
        sc_ref[1] += logsig_sum(-dn)
        s_ref[...] += (colsum(wz.astype(jnp.bfloat16), zb)
                       + colsum(wp.astype(jnp.bfloat16), zpb)
                       + colsum(wn.astype(jnp.bfloat16), znb))
        sc_ref[2] += jnp.sum(wz) + jnp.sum(wp) + jnp.sum(wn)

    @pl.when(g >= NZ)
    def _phase_b():
        mu = s_ref[...] / jnp.maximum(sc_ref[2], 1.0)
        m = jnp.sum(mu * mu)
        mu2b = (mu * -2.0).astype(jnp.bfloat16)

        def contrib(xb, w):
            q = lanered(ones_row, xb * xb)
            r = lanered(mu2b, xb)
            return jnp.sum(jnp.sqrt(jnp.maximum(w * (q + r + m), 0.0)))

        sc_ref[3] += contrib(zb, wz) + contrib(zpb, wp) + contrib(znb, wn)

    @pl.when(g == 2 * NZ - 1)
    def _fin():
        sig = 1.0 / (1.0 + jnp.exp(-sc_ref[3]))
        total = -(sc_ref[0] + sc_ref[1]) / N + sig - 0.5
        o_ref[...] = jnp.full((1, 1), total, dtype=jnp.float32)


def kernel(out, mask):
    w = mask.astype(jnp.float32).reshape(3 * NZ, 1, B)

    def omap(t):
        return lambda g: (t * NZ + g % NZ, 0)

    def wmap(t):
        return lambda g: (t * NZ + g % NZ, 0, 0)

    res = pl.pallas_call(
        _body,
        grid=(2 * NZ,),
        in_specs=[
            pl.BlockSpec((B, D), omap(0)),
            pl.BlockSpec((B, D), omap(1)),
            pl.BlockSpec((B, D), omap(2)),
            pl.BlockSpec((1, 1, B), wmap(0)),
            pl.BlockSpec((1, 1, B), wmap(1)),
            pl.BlockSpec((1, 1, B), wmap(2)),
        ],
        out_specs=pl.BlockSpec((1, 1), lambda g: (0, 0)),
        out_shape=jax.ShapeDtypeStruct((1, 1), jnp.float32),
        scratch_shapes=[
            pltpu.VMEM((1, D), jnp.float32),
            pltpu.SMEM((4,), jnp.float32),
        ],
        compiler_params=pltpu.CompilerParams(
            dimension_semantics=("arbitrary",),
        ),
    )(out, out, out, w, w, w)
    return res[0, 0]
